# baseline (speedup 1.0000x reference)
import functools

import jax
import jax.numpy as jnp
from jax import lax
from jax.experimental import pallas as pl
from jax.experimental.pallas import tpu as pltpu
from jax.experimental.pallas import tpu_sc as plsc

B, H, N = 32, 16, 512
NN = N * N
TABLE = 3375
PITCH = 17

NC, NS = 2, 16
NW = NC * NS
PER_W = NN // NW


HB = 8


def _sc_gather(table_flat, idx16):
    mesh = plsc.VectorSubcoreMesh(core_axis_name="c", subcore_axis_name="s")

    CH = 1024
    NCH = PER_W // CH

    CHG = CH // 128

    @functools.partial(
        pl.kernel,
        out_type=jax.ShapeDtypeStruct((H, NN // 128, 128), jnp.float32),
        mesh=mesh,
        scratch_types=[
            pltpu.VMEM((TABLE * PITCH + 1,), jnp.float32),
            pltpu.VMEM((PER_W,), jnp.int32),
            pltpu.VMEM((H, CHG, 128), jnp.float32),
            pltpu.VMEM((H, CHG, 128), jnp.float32),
            pltpu.SemaphoreType.DMA,
        ],
        compiler_params=pltpu.CompilerParams(needs_layout_passes=False),
    )
    def k(table_hbm, idx_hbm, out_hbm, table_v, idx_v, buf_a, buf_b, sem):
        wid = lax.axis_index("s") * NC + lax.axis_index("c")
        base = wid * PER_W
        pltpu.sync_copy(table_hbm, table_v)
        pltpu.sync_copy(idx_hbm.at[pl.ds(base, PER_W)], idx_v)
        bufs = [buf_a, buf_b]
        pending = [[], []]
        for c in range(NCH):
            buf = bufs[c % 2]
            for cp in pending[c % 2]:
                cp.wait()
            pending[c % 2] = []

            @plsc.parallel_loop(0, CH // 16, unroll=4)
            def body(v, c=c, buf=buf):
                g = v // 8
                l0 = (v % 8) * 16
                ids = idx_v[pl.ds(c * CH + v * 16, 16)]
                for h in range(H):
                    buf[h, g, pl.ds(l0, 16)] = plsc.load_gather(
                        table_v, [ids + h])

            gbase = wid * (PER_W // 128) + c * CHG
            pending[c % 2].append(pltpu.async_copy(
                buf,
                out_hbm.at[:, pl.ds(gbase, CHG), :],
                sem))
        for cps in pending:
            for cp in cps:
                cp.wait()

    return k(table_flat, idx16)


def _add_body(a_ref, b_ref, o_ref, scr_ref):
    @pl.when(pl.program_id(1) == 0)
    def _():
        scr_ref[...] = b_ref[...].reshape(HB, N, N)

    o_ref[0] = a_ref[0] + scr_ref[...]


def _broadcast_add(attn, bias3):
    return pl.pallas_call(
        _add_body,
        grid=(H // HB, B),
        in_specs=[
            pl.BlockSpec((1, HB, N, N), lambda h, b: (b, h, 0, 0)),
            pl.BlockSpec((HB, NN // 128, 128), lambda h, b: (h, 0, 0)),
        ],
        out_specs=pl.BlockSpec((1, HB, N, N), lambda h, b: (b, h, 0, 0)),
        out_shape=jax.ShapeDtypeStruct((B, H, N, N), jnp.float32),
        scratch_shapes=[pltpu.VMEM((HB, N, N), jnp.float32)],
    )(attn, bias3)


def kernel(attn, relative_position_bias_table, relative_position_index, n):
    idxp = (relative_position_index.reshape(-1) + (n - N)) * PITCH
    idxp = idxp.astype(jnp.int32)
    table_pad = jnp.pad(relative_position_bias_table, ((0, 0), (0, PITCH - H)))
    table_flat = jnp.pad(table_pad.reshape(-1), (0, 1))
    bias = _sc_gather(table_flat, idxp)
    return _broadcast_add(attn, bias)


# speedup vs baseline: 1.0026x; 1.0026x over previous
import functools

import jax
import jax.numpy as jnp
from jax import lax
from jax.experimental import pallas as pl
from jax.experimental.pallas import tpu as pltpu
from jax.experimental.pallas import tpu_sc as plsc

B, H, N = 32, 16, 512
NN = N * N
TABLE = 3375
PITCH = 17

NC, NS = 2, 16
NW = NC * NS
PER_W = NN // NW


HB = 8


def _sc_gather(table_flat, idx16):
    mesh = plsc.VectorSubcoreMesh(core_axis_name="c", subcore_axis_name="s")

    CH = 1024
    NCH = PER_W // CH

    CHG = CH // 128

    @functools.partial(
        pl.kernel,
        out_type=jax.ShapeDtypeStruct((H, NN // 128, 128), jnp.float32),
        mesh=mesh,
        scratch_types=[
            pltpu.VMEM((TABLE * PITCH + 1,), jnp.float32),
            pltpu.VMEM((PER_W,), jnp.int32),
            pltpu.VMEM((H, CHG, 128), jnp.float32),
            pltpu.VMEM((H, CHG, 128), jnp.float32),
            pltpu.SemaphoreType.DMA,
        ],
        compiler_params=pltpu.CompilerParams(needs_layout_passes=False),
    )
    def k(table_hbm, idx_hbm, out_hbm, table_v, idx_v, buf_a, buf_b, sem):
        wid = lax.axis_index("s") * NC + lax.axis_index("c")
        base = wid * PER_W
        pltpu.sync_copy(table_hbm, table_v)
        pltpu.sync_copy(idx_hbm.at[pl.ds(base, PER_W)], idx_v)
        bufs = [buf_a, buf_b]
        pending = [[], []]
        for c in range(NCH):
            buf = bufs[c % 2]
            for cp in pending[c % 2]:
                cp.wait()
            pending[c % 2] = []

            @plsc.parallel_loop(0, CH // 16, unroll=2)
            def body(v, c=c, buf=buf):
                g = v // 8
                l0 = (v % 8) * 16
                ids = idx_v[pl.ds(c * CH + v * 16, 16)]
                for h in range(H):
                    buf[h, g, pl.ds(l0, 16)] = plsc.load_gather(
                        table_v, [ids + h])

            gbase = wid * (PER_W // 128) + c * CHG
            pending[c % 2].append(pltpu.async_copy(
                buf,
                out_hbm.at[:, pl.ds(gbase, CHG), :],
                sem))
        for cps in pending:
            for cp in cps:
                cp.wait()

    return k(table_flat, idx16)


def _add_body(a_ref, b_ref, o_ref, scr_ref):
    @pl.when(pl.program_id(1) == 0)
    def _():
        scr_ref[...] = b_ref[...].reshape(HB, N, N)

    o_ref[0] = a_ref[0] + scr_ref[...]


def _broadcast_add(attn, bias3):
    return pl.pallas_call(
        _add_body,
        grid=(H // HB, B),
        in_specs=[
            pl.BlockSpec((1, HB, N, N), lambda h, b: (b, h, 0, 0)),
            pl.BlockSpec((HB, NN // 128, 128), lambda h, b: (h, 0, 0)),
        ],
        out_specs=pl.BlockSpec((1, HB, N, N), lambda h, b: (b, h, 0, 0)),
        out_shape=jax.ShapeDtypeStruct((B, H, N, N), jnp.float32),
        scratch_shapes=[pltpu.VMEM((HB, N, N), jnp.float32)],
    )(attn, bias3)


def kernel(attn, relative_position_bias_table, relative_position_index, n):
    idxp = (relative_position_index.reshape(-1) + (n - N)) * PITCH
    idxp = idxp.astype(jnp.int32)
    table_pad = jnp.pad(relative_position_bias_table, ((0, 0), (0, PITCH - H)))
    table_flat = jnp.pad(table_pad.reshape(-1), (0, 1))
    bias = _sc_gather(table_flat, idxp)
    return _broadcast_add(attn, bias)


# speedup vs baseline: 1.0027x; 1.0001x over previous
import functools

import jax
import jax.numpy as jnp
from jax import lax
from jax.experimental import pallas as pl
from jax.experimental.pallas import tpu as pltpu
from jax.experimental.pallas import tpu_sc as plsc

B, H, N = 32, 16, 512
NN = N * N
TABLE = 3375
PITCH = 17

NC, NS = 2, 16
NW = NC * NS
PER_W = NN // NW


HB = 8


def _sc_gather(table_flat, idxp):
    mesh = plsc.VectorSubcoreMesh(core_axis_name="c", subcore_axis_name="s")

    CH = 1024
    NCH = PER_W // CH

    CHG = CH // 128

    @functools.partial(
        pl.kernel,
        out_type=jax.ShapeDtypeStruct((H, NN // 128, 128), jnp.float32),
        mesh=mesh,
        scratch_types=[
            pltpu.VMEM((TABLE * PITCH + 1,), jnp.float32),
            pltpu.VMEM((PER_W,), jnp.int32),
            pltpu.VMEM((H, CHG, 128), jnp.float32),
            pltpu.VMEM((H, CHG, 128), jnp.float32),
            pltpu.SemaphoreType.DMA,
        ],
        compiler_params=pltpu.CompilerParams(needs_layout_passes=False),
    )
    def k(table_hbm, idx_hbm, out_hbm, table_v, idx_v, buf_a, buf_b, sem):
        wid = lax.axis_index("s") * NC + lax.axis_index("c")
        base = wid * PER_W
        pltpu.sync_copy(table_hbm, table_v)
        pltpu.sync_copy(idx_hbm.at[pl.ds(base, PER_W)], idx_v)
        bufs = [buf_a, buf_b]
        pending = [[], []]
        for c in range(NCH):
            buf = bufs[c % 2]
            for cp in pending[c % 2]:
                cp.wait()
            pending[c % 2] = []

            @plsc.parallel_loop(0, CH // 16, unroll=2)
            def body(v, c=c, buf=buf):
                g = v // 8
                l0 = (v % 8) * 16
                ids = idx_v[pl.ds(c * CH + v * 16, 16)]
                for h in range(H):
                    buf[h, g, pl.ds(l0, 16)] = plsc.load_gather(
                        table_v, [ids + h])

            gbase = wid * (PER_W // 128) + c * CHG
            pending[c % 2].append(pltpu.async_copy(
                buf,
                out_hbm.at[:, pl.ds(gbase, CHG), :],
                sem))
        for cps in pending:
            for cp in cps:
                cp.wait()

    return k(table_flat, idxp)


def _add_body(a_ref, b_ref, o_ref, scr_ref):
    @pl.when(pl.program_id(1) == 0)
    def _():
        scr_ref[...] = b_ref[...].reshape(HB, N, N)

    o_ref[0] = a_ref[0] + scr_ref[...]


def _broadcast_add(attn, bias3):
    return pl.pallas_call(
        _add_body,
        grid=(H // HB, B),
        in_specs=[
            pl.BlockSpec((1, HB, N, N), lambda h, b: (b, h, 0, 0)),
            pl.BlockSpec((HB, NN // 128, 128), lambda h, b: (h, 0, 0)),
        ],
        out_specs=pl.BlockSpec((1, HB, N, N), lambda h, b: (b, h, 0, 0)),
        out_shape=jax.ShapeDtypeStruct((B, H, N, N), jnp.float32),
        scratch_shapes=[pltpu.VMEM((HB, N, N), jnp.float32)],
    )(attn, bias3)


def kernel(attn, relative_position_bias_table, relative_position_index, n):
    idxp = (relative_position_index.reshape(-1) + (n - N)) * PITCH
    idxp = idxp.astype(jnp.int32)
    table_pad = jnp.pad(relative_position_bias_table, ((0, 0), (0, PITCH - H)))
    table_flat = jnp.pad(table_pad.reshape(-1), (0, 1))
    bias = _sc_gather(table_flat, idxp)
    return _broadcast_add(attn, bias)
